# baseline (speedup 1.0000x reference)
import jax
import jax.numpy as jnp
from jax.experimental import pallas as pl
from jax.experimental.pallas import tpu as pltpu

_BR = 1536


def _body(mask_ref, x_ref, o_ref):
    xv = x_ref[...]
    mask = mask_ref[...] != 0
    binar = (xv >= 0.0).astype(xv.dtype)
    o_ref[...] = jnp.where(mask, binar, xv)


def kernel(x, col_idxs):
    m, n = x.shape
    mask = jnp.zeros((1, n), jnp.int32).at[0, col_idxs].set(1)
    grid = (pl.cdiv(m, _BR),)
    return pl.pallas_call(
        _body,
        grid=grid,
        in_specs=[
            pl.BlockSpec((1, n), lambda i: (0, 0)),
            pl.BlockSpec((_BR, n), lambda i: (i, 0)),
        ],
        out_specs=pl.BlockSpec((_BR, n), lambda i: (i, 0)),
        out_shape=jax.ShapeDtypeStruct((m, n), x.dtype),
        compiler_params=pltpu.CompilerParams(
            dimension_semantics=("parallel",),
        ),
    )(mask, x)


# speedup vs baseline: 1.0025x; 1.0025x over previous
import jax
import jax.numpy as jnp
from jax.experimental import pallas as pl
from jax.experimental.pallas import tpu as pltpu

_BR = 1920


def _body(mask_ref, x_ref, o_ref):
    xv = x_ref[...]
    mask = mask_ref[...] != 0
    binar = (xv >= 0.0).astype(xv.dtype)
    o_ref[...] = jnp.where(mask, binar, xv)


def kernel(x, col_idxs):
    m, n = x.shape
    mask = jnp.zeros((1, n), jnp.int32).at[0, col_idxs].set(1)
    grid = (pl.cdiv(m, _BR),)
    return pl.pallas_call(
        _body,
        grid=grid,
        in_specs=[
            pl.BlockSpec((1, n), lambda i: (0, 0)),
            pl.BlockSpec((_BR, n), lambda i: (i, 0)),
        ],
        out_specs=pl.BlockSpec((_BR, n), lambda i: (i, 0)),
        out_shape=jax.ShapeDtypeStruct((m, n), x.dtype),
        compiler_params=pltpu.CompilerParams(
            dimension_semantics=("parallel",),
            vmem_limit_bytes=67108864,

        ),
    )(mask, x)
